# baseline (speedup 1.0000x reference)
import functools

import jax
import jax.numpy as jnp
from jax import lax
from jax.experimental import pallas as pl
from jax.experimental.pallas import tpu as pltpu
from jax.experimental.pallas import tpu_sc as plsc

B = 4096
D = 16
F = 26
SV = 100000
IN_FEAT = (F + 2) * D

NC = 2
NS = 16
NW = NC * NS
BPW = B // NW

@functools.cache
def _sc_gather_fn():
    mesh = plsc.VectorSubcoreMesh(core_axis_name="c", subcore_axis_name="s")

    @functools.partial(
        pl.kernel,
        out_type=(
            jax.ShapeDtypeStruct((B, D), jnp.float32),
            jax.ShapeDtypeStruct((B, D), jnp.float32),
            jax.ShapeDtypeStruct((NW, F, BPW, D), jnp.float32),
        ),
        mesh=mesh,
        compiler_params=pltpu.CompilerParams(use_tc_tiling_on_sc=False),
        scratch_types=[
            pltpu.VMEM((BPW,), jnp.int32),
            pltpu.VMEM((BPW,), jnp.int32),
            pltpu.VMEM((F, BPW), jnp.int32),
            pltpu.VMEM((BPW, D), jnp.float32),
            pltpu.VMEM((BPW, D), jnp.float32),
            pltpu.VMEM((F, BPW, D), jnp.float32),
            pltpu.SemaphoreType.DMA,
        ],
    )
    def _sc_gather(ut, it, sflat, uid, iid, spid, ue_o, ie_o, sp_o,
                   uidx_v, iidx_v, spidx_v, ur_v, ir_v, spr_v, sem):
        wid = lax.axis_index("s") * NC + lax.axis_index("c")
        base = wid * BPW
        pltpu.sync_copy(uid.at[pl.ds(base, BPW)], uidx_v)
        pltpu.sync_copy(iid.at[pl.ds(base, BPW)], iidx_v)
        pltpu.sync_copy(spid.at[wid], spidx_v)
        copies = [
            pltpu.async_copy(ut.at[uidx_v], ur_v, sem),
            pltpu.async_copy(it.at[iidx_v], ir_v, sem),
        ]
        for j in range(F):
            copies.append(pltpu.async_copy(sflat.at[spidx_v.at[j]], spr_v.at[j], sem))
        for c in copies:
            c.wait()
        pltpu.sync_copy(ur_v, ue_o.at[pl.ds(base, BPW)])
        pltpu.sync_copy(ir_v, ie_o.at[pl.ds(base, BPW)])
        pltpu.sync_copy(spr_v, sp_o.at[wid])

    return _sc_gather


def _dense_body(ue, ie, sp, K, cb, W0, b0, W1, b1, W2, b2, Wo, bo, Wt, out):
    x0 = jnp.concatenate([ue[...], ie[...], sp[...]], axis=1)
    dn = (((1,), (1,)), ((), ()))
    dot = lax.dot_general(x0, K[0], dn, preferred_element_type=jnp.float32)
    x1 = x0 * (dot + cb[0][None, :]) + x0
    dot = lax.dot_general(x1, K[1], dn, preferred_element_type=jnp.float32)
    x2 = x0 * (dot + cb[1][None, :]) + x1
    h = jnp.maximum(jnp.dot(x0, W0[...], preferred_element_type=jnp.float32) + b0[...], 0.0)
    h = jnp.maximum(jnp.dot(h, W1[...], preferred_element_type=jnp.float32) + b1[...], 0.0)
    h = jnp.maximum(jnp.dot(h, W2[...], preferred_element_type=jnp.float32) + b2[...], 0.0)
    deep = jnp.dot(h, Wo[...], preferred_element_type=jnp.float32) + bo[...]
    logit = (jnp.dot(x2, Wt[:IN_FEAT, :], preferred_element_type=jnp.float32)
             + jnp.dot(deep, Wt[IN_FEAT:, :], preferred_element_type=jnp.float32))
    out[...] = 1.0 / (1.0 + jnp.exp(-logit))


def _dense_call(ue, ie, sp, K, cb, W0, b0, W1, b1, W2, b2, Wo, bo, Wt):
    BB = 1024
    grid = (B // BB,)
    full = lambda *s: pl.BlockSpec(s, lambda i: (0,) * len(s))
    return pl.pallas_call(
        _dense_body,
        grid=grid,
        in_specs=[
            pl.BlockSpec((BB, D), lambda i: (i, 0)),
            pl.BlockSpec((BB, D), lambda i: (i, 0)),
            pl.BlockSpec((BB, F * D), lambda i: (i, 0)),
            full(2, IN_FEAT, IN_FEAT),
            full(2, IN_FEAT),
            full(IN_FEAT, 2 * D),
            full(1, 2 * D),
            full(2 * D, 2 * D),
            full(1, 2 * D),
            full(2 * D, 2 * D),
            full(1, 2 * D),
            full(2 * D, D),
            full(1, D),
            full(IN_FEAT + D, 1),
        ],
        out_specs=pl.BlockSpec((BB, 1), lambda i: (i, 0)),
        out_shape=jax.ShapeDtypeStruct((B, 1), jnp.float32),
    )(ue, ie, sp, K, cb, W0, b0, W1, b1, W2, b2, Wo, bo, Wt)


def kernel(user_ids, item_ids, sparse_features, user_table, item_table,
           sparse_tables, kernels, cbias, W0, b0, W1, b1, W2, b2, Wo, bo, Wt):
    sflat = sparse_tables.reshape(F * SV, D)
    spid = (sparse_features.astype(jnp.int32)
            + (jnp.arange(F, dtype=jnp.int32) * SV)[None, :]).reshape(NW, F, BPW)
    ue, ie, sp = _sc_gather_fn()(user_table, item_table, sflat,
                                 user_ids.astype(jnp.int32), item_ids.astype(jnp.int32), spid)
    sp = sp.reshape(B, F * D)
    return _dense_call(
        ue, ie, sp, kernels, cbias.reshape(2, IN_FEAT),
        W0, b0.reshape(1, 2 * D), W1, b1.reshape(1, 2 * D),
        W2, b2.reshape(1, 2 * D), Wo, bo.reshape(1, D), Wt)
